# Initial kernel scaffold; baseline (speedup 1.0000x reference)
import jax
import jax.numpy as jnp
from jax.experimental import pallas as pl


def kernel(x, edge_index, W1, b1, W2, b2, W_lin, b_lin):
    raise NotImplementedError("write your pallas kernel here")


# speedup vs baseline: 9.7438x; 9.7438x over previous
import functools

import jax
import jax.numpy as jnp
from jax import lax
from jax.experimental import pallas as pl
from jax.experimental.pallas import tpu as pltpu
from jax.experimental.pallas import tpu_sc as plsc

N = 10000
E = 320000
F = 128
NW = 32
CH = 80
EPT = CH * 128
GCH = 16
GW = CH // GCH
EP = NW * EPT
NR = 80
AR = NR * 128
STRIPE = AR // 16
RSTRIPE = NR // 16

_mesh = plsc.VectorSubcoreMesh(core_axis_name="c", subcore_axis_name="s",
                               num_cores=2, num_subcores=16)


@functools.partial(
    pl.kernel,
    out_type=jax.ShapeDtypeStruct((2, AR), jnp.float32),
    mesh=_mesh,
    scratch_types=[
        pltpu.VMEM((GW, GCH, 128), jnp.int32),
        pltpu.VMEM((128,), jnp.float32),
        pltpu.VMEM_SHARED((AR,), jnp.float32),
    ],
)
def _deg_kernel(dst_hbm, zeros_hbm, out_hbm, dst_v, ones_v, acc_sh):
    cid = lax.axis_index("c")
    sid = lax.axis_index("s")
    wid = cid * 16 + sid
    pltpu.sync_copy(dst_hbm.at[wid], dst_v)
    for k in range(8):
        ones_v[pl.ds(k * 16, 16)] = jnp.ones((16,), jnp.float32)
    pltpu.sync_copy(zeros_hbm.at[pl.ds(sid * STRIPE, STRIPE)],
                    acc_sh.at[pl.ds(sid * STRIPE, STRIPE)])
    plsc.subcore_barrier()

    def body(j, _):
        pltpu.sync_copy(ones_v, acc_sh.at[dst_v.at[j // GCH, j % GCH]],
                        add=True)
        return _

    lax.fori_loop(0, CH, body, 0)
    plsc.subcore_barrier()
    pltpu.sync_copy(acc_sh.at[pl.ds(sid * STRIPE, STRIPE)],
                    out_hbm.at[cid, pl.ds(sid * STRIPE, STRIPE)])


@functools.partial(
    pl.kernel,
    out_type=jax.ShapeDtypeStruct((2, AR, F), jnp.float32),
    mesh=_mesh,
    scratch_types=[
        pltpu.VMEM((2, GCH, 128), jnp.int32),
        pltpu.VMEM((2, GCH, 128), jnp.int32),
        pltpu.VMEM((2, 128, F), jnp.float32),
        pltpu.VMEM_SHARED((AR, F), jnp.float32),
        pltpu.SemaphoreType.DMA,
        pltpu.SemaphoreType.DMA,
        pltpu.SemaphoreType.DMA,
    ],
)
def _agg_kernel(g_hbm, src_hbm, dst_hbm, zeros_hbm, out_hbm,
                src_v, dst_v, rows_v, acc_sh, sem0, sem1, semi):
    cid = lax.axis_index("c")
    sid = lax.axis_index("s")
    wid = cid * 16 + sid
    pltpu.sync_copy(src_hbm.at[wid, 0], src_v.at[0])
    pltpu.sync_copy(dst_hbm.at[wid, 0], dst_v.at[0])
    pltpu.async_copy(src_hbm.at[wid, 1], src_v.at[1], semi)
    pltpu.async_copy(dst_hbm.at[wid, 1], dst_v.at[1], semi)
    pltpu.async_copy(g_hbm.at[src_v.at[0, 0]], rows_v.at[0], sem0)
    pltpu.async_copy(g_hbm.at[src_v.at[0, 1]], rows_v.at[1], sem1)
    pltpu.sync_copy(zeros_hbm.at[pl.ds(sid * STRIPE, STRIPE)],
                    acc_sh.at[pl.ds(sid * STRIPE, STRIPE)])
    plsc.subcore_barrier()

    for g in range(GW):
        b = g % 2

        def body(jj, carry, b=b):
            j0 = 2 * jj
            pltpu.make_async_copy(g_hbm.at[src_v.at[b, j0]],
                                  rows_v.at[0], sem0).wait()
            pltpu.sync_copy(rows_v.at[0], acc_sh.at[dst_v.at[b, j0]], add=True)

            @pl.when(j0 + 2 < GCH)
            def _pf0():
                pltpu.async_copy(g_hbm.at[src_v.at[b, j0 + 2]],
                                 rows_v.at[0], sem0)

            pltpu.make_async_copy(g_hbm.at[src_v.at[b, j0 + 1]],
                                  rows_v.at[1], sem1).wait()
            pltpu.sync_copy(rows_v.at[1], acc_sh.at[dst_v.at[b, j0 + 1]],
                            add=True)

            @pl.when(j0 + 3 < GCH)
            def _pf1():
                pltpu.async_copy(g_hbm.at[src_v.at[b, j0 + 3]],
                                 rows_v.at[1], sem1)

            return carry

        lax.fori_loop(0, GCH // 2, body, 0)
        if g + 1 < GW:
            nb = (g + 1) % 2
            pltpu.make_async_copy(src_hbm.at[wid, g + 1], src_v.at[nb],
                                  semi).wait()
            pltpu.make_async_copy(dst_hbm.at[wid, g + 1], dst_v.at[nb],
                                  semi).wait()
            if g + 2 < GW:
                pltpu.async_copy(src_hbm.at[wid, g + 2], src_v.at[b], semi)
                pltpu.async_copy(dst_hbm.at[wid, g + 2], dst_v.at[b], semi)
            pltpu.async_copy(g_hbm.at[src_v.at[nb, 0]], rows_v.at[0], sem0)
            pltpu.async_copy(g_hbm.at[src_v.at[nb, 1]], rows_v.at[1], sem1)

    plsc.subcore_barrier()
    pltpu.sync_copy(acc_sh.at[pl.ds(sid * STRIPE, STRIPE)],
                    out_hbm.at[cid, pl.ds(sid * STRIPE, STRIPE)])


def _dinv_body(deg_ref, o_ref):
    deg = deg_ref[0] + deg_ref[1] + 1.0
    o_ref[...] = lax.rsqrt(deg)


_dinv_call = pl.pallas_call(
    _dinv_body,
    out_shape=jax.ShapeDtypeStruct((NR, 128), jnp.float32),
)

_BLK = 1000
_GRID = N // _BLK


def _scale_mm_body(x_ref, w_ref, dinv_ref, g_ref):
    h = jnp.dot(x_ref[...], w_ref[...], preferred_element_type=jnp.float32)
    g_ref[...] = h * dinv_ref[...]


_scale_mm_call = pl.pallas_call(
    _scale_mm_body,
    grid=(_GRID,),
    in_specs=[
        pl.BlockSpec((_BLK, F), lambda i: (i, 0)),
        pl.BlockSpec((F, F), lambda i: (0, 0)),
        pl.BlockSpec((_BLK, 1), lambda i: (i, 0)),
    ],
    out_specs=pl.BlockSpec((_BLK, F), lambda i: (i, 0)),
    out_shape=jax.ShapeDtypeStruct((N, F), jnp.float32),
)


def _layer_body(p_ref, g1_ref, dinv_ref, b1_ref, w2_ref, x1_ref, g2_ref):
    t = (p_ref[0] + p_ref[1] + g1_ref[...]) * dinv_ref[...] + b1_ref[...]
    x1 = jnp.maximum(t, 0.0)
    x1_ref[...] = x1
    h2 = jnp.dot(x1, w2_ref[...], preferred_element_type=jnp.float32)
    g2_ref[...] = h2 * dinv_ref[...]


_layer_call = pl.pallas_call(
    _layer_body,
    grid=(_GRID,),
    in_specs=[
        pl.BlockSpec((2, _BLK, F), lambda i: (0, i, 0)),
        pl.BlockSpec((_BLK, F), lambda i: (i, 0)),
        pl.BlockSpec((_BLK, 1), lambda i: (i, 0)),
        pl.BlockSpec((1, F), lambda i: (0, 0)),
        pl.BlockSpec((F, F), lambda i: (0, 0)),
    ],
    out_specs=[
        pl.BlockSpec((_BLK, F), lambda i: (i, 0)),
        pl.BlockSpec((_BLK, F), lambda i: (i, 0)),
    ],
    out_shape=[
        jax.ShapeDtypeStruct((N, F), jnp.float32),
        jax.ShapeDtypeStruct((N, F), jnp.float32),
    ],
)


def _final_body(q_ref, g2_ref, dinv_ref, b2_ref, x1_ref, wt_ref, wb_ref,
                bl_ref, o_ref):
    t = (q_ref[0] + q_ref[1] + g2_ref[...]) * dinv_ref[...] + b2_ref[...]
    x2 = jnp.maximum(t, 0.0)
    o_ref[...] = (
        jnp.dot(x1_ref[...], wt_ref[...], preferred_element_type=jnp.float32)
        + jnp.dot(x2, wb_ref[...], preferred_element_type=jnp.float32)
        + bl_ref[...]
    )


_final_call = pl.pallas_call(
    _final_body,
    grid=(_GRID,),
    in_specs=[
        pl.BlockSpec((2, _BLK, F), lambda i: (0, i, 0)),
        pl.BlockSpec((_BLK, F), lambda i: (i, 0)),
        pl.BlockSpec((_BLK, 1), lambda i: (i, 0)),
        pl.BlockSpec((1, F), lambda i: (0, 0)),
        pl.BlockSpec((_BLK, F), lambda i: (i, 0)),
        pl.BlockSpec((F, F), lambda i: (0, 0)),
        pl.BlockSpec((F, F), lambda i: (0, 0)),
        pl.BlockSpec((1, F), lambda i: (0, 0)),
    ],
    out_specs=pl.BlockSpec((_BLK, F), lambda i: (i, 0)),
    out_shape=jax.ShapeDtypeStruct((N, F), jnp.float32),
)


def kernel(x, edge_index, W1, b1, W2, b2, W_lin, b_lin):
    src = edge_index[0].astype(jnp.int32)
    dst = edge_index[1].astype(jnp.int32)
    pad = EP - E
    src3 = jnp.concatenate([src, jnp.zeros((pad,), jnp.int32)]).reshape(NW, GW, GCH, 128)
    dst_pad = N + (jnp.arange(pad, dtype=jnp.int32) % (AR - N))
    dst3 = jnp.concatenate([dst, dst_pad]).reshape(NW, GW, GCH, 128)
    zeros2d = jnp.zeros((AR, F), jnp.float32)
    zeros_deg = jnp.zeros((AR,), jnp.float32)

    deg_parts = _deg_kernel(dst3, zeros_deg)
    dinv2d = _dinv_call(deg_parts.reshape(2, NR, 128))
    dinv_col = dinv2d.reshape(-1)[:N, None]

    g1 = _scale_mm_call(x, W1, dinv_col)
    p = _agg_kernel(g1, src3, dst3, zeros2d)
    x1, g2 = _layer_call(p, g1, dinv_col, b1.reshape(1, F), W2)
    q = _agg_kernel(g2, src3, dst3, zeros2d)
    out = _final_call(q, g2, dinv_col, b2.reshape(1, F), x1,
                      W_lin[:F], W_lin[F:], b_lin.reshape(1, F))
    return out


# speedup vs baseline: 11.1758x; 1.1470x over previous
import functools

import jax
import jax.numpy as jnp
from jax import lax
from jax.experimental import pallas as pl
from jax.experimental.pallas import tpu as pltpu
from jax.experimental.pallas import tpu_sc as plsc

N = 10000
E = 320000
F = 128
NW = 32
CH = 80
EPT = CH * 128
GCH = 16
GW = CH // GCH
EP = NW * EPT
NR = 80
AR = NR * 128
STRIPE = AR // 16
RSTRIPE = NR // 16

_mesh = plsc.VectorSubcoreMesh(core_axis_name="c", subcore_axis_name="s",
                               num_cores=2, num_subcores=16)


@functools.partial(
    pl.kernel,
    out_type=jax.ShapeDtypeStruct((2, AR), jnp.float32),
    mesh=_mesh,
    scratch_types=[
        pltpu.VMEM((GW, GCH, 128), jnp.int32),
        pltpu.VMEM((128,), jnp.float32),
        pltpu.VMEM_SHARED((AR,), jnp.float32),
    ],
)
def _deg_kernel(dst_hbm, zeros_hbm, out_hbm, dst_v, ones_v, acc_sh):
    cid = lax.axis_index("c")
    sid = lax.axis_index("s")
    wid = cid * 16 + sid
    pltpu.sync_copy(dst_hbm.at[wid], dst_v)
    for k in range(8):
        ones_v[pl.ds(k * 16, 16)] = jnp.ones((16,), jnp.float32)
    pltpu.sync_copy(zeros_hbm.at[pl.ds(sid * STRIPE, STRIPE)],
                    acc_sh.at[pl.ds(sid * STRIPE, STRIPE)])
    plsc.subcore_barrier()

    def body(j, _):
        pltpu.sync_copy(ones_v, acc_sh.at[dst_v.at[j // GCH, j % GCH]],
                        add=True)
        return _

    lax.fori_loop(0, CH, body, 0)
    plsc.subcore_barrier()
    pltpu.sync_copy(acc_sh.at[pl.ds(sid * STRIPE, STRIPE)],
                    out_hbm.at[cid, pl.ds(sid * STRIPE, STRIPE)])


@functools.partial(
    pl.kernel,
    out_type=jax.ShapeDtypeStruct((2, AR, F), jnp.float32),
    mesh=_mesh,
    scratch_types=[
        pltpu.VMEM((2, GCH, 128), jnp.int32),
        pltpu.VMEM((2, GCH, 128), jnp.int32),
        pltpu.VMEM((2, 128, F), jnp.float32),
        pltpu.VMEM_SHARED((AR, F), jnp.float32),
        pltpu.SemaphoreType.DMA,
        pltpu.SemaphoreType.DMA,
        pltpu.SemaphoreType.DMA,
    ],
)
def _agg_kernel(g_hbm, src_hbm, dst_hbm, zeros_hbm, out_hbm,
                src_v, dst_v, rows_v, acc_sh, sem0, sem1, semi):
    cid = lax.axis_index("c")
    sid = lax.axis_index("s")
    wid = cid * 16 + sid
    pltpu.sync_copy(src_hbm.at[wid, 0], src_v.at[0])
    pltpu.sync_copy(dst_hbm.at[wid, 0], dst_v.at[0])
    pltpu.async_copy(src_hbm.at[wid, 1], src_v.at[1], semi)
    pltpu.async_copy(dst_hbm.at[wid, 1], dst_v.at[1], semi)
    pltpu.async_copy(g_hbm.at[src_v.at[0, 0]], rows_v.at[0], sem0)
    pltpu.async_copy(g_hbm.at[src_v.at[0, 1]], rows_v.at[1], sem1)
    pltpu.sync_copy(zeros_hbm.at[pl.ds(sid * STRIPE, STRIPE)],
                    acc_sh.at[pl.ds(sid * STRIPE, STRIPE)])
    plsc.subcore_barrier()

    for g in range(GW):
        b = g % 2

        def body(jj, carry, b=b):
            j0 = 2 * jj
            pltpu.make_async_copy(g_hbm.at[src_v.at[b, j0]],
                                  rows_v.at[0], sem0).wait()
            pltpu.sync_copy(rows_v.at[0], acc_sh.at[dst_v.at[b, j0]], add=True)

            @pl.when(j0 + 2 < GCH)
            def _pf0():
                pltpu.async_copy(g_hbm.at[src_v.at[b, j0 + 2]],
                                 rows_v.at[0], sem0)

            pltpu.make_async_copy(g_hbm.at[src_v.at[b, j0 + 1]],
                                  rows_v.at[1], sem1).wait()
            pltpu.sync_copy(rows_v.at[1], acc_sh.at[dst_v.at[b, j0 + 1]],
                            add=True)

            @pl.when(j0 + 3 < GCH)
            def _pf1():
                pltpu.async_copy(g_hbm.at[src_v.at[b, j0 + 3]],
                                 rows_v.at[1], sem1)

            return carry

        lax.fori_loop(0, GCH // 2, body, 0)
        if g + 1 < GW:
            nb = (g + 1) % 2
            pltpu.make_async_copy(src_hbm.at[wid, g + 1], src_v.at[nb],
                                  semi).wait()
            pltpu.make_async_copy(dst_hbm.at[wid, g + 1], dst_v.at[nb],
                                  semi).wait()
            if g + 2 < GW:
                pltpu.async_copy(src_hbm.at[wid, g + 2], src_v.at[b], semi)
                pltpu.async_copy(dst_hbm.at[wid, g + 2], dst_v.at[b], semi)
            pltpu.async_copy(g_hbm.at[src_v.at[nb, 0]], rows_v.at[0], sem0)
            pltpu.async_copy(g_hbm.at[src_v.at[nb, 1]], rows_v.at[1], sem1)

    plsc.subcore_barrier()
    pltpu.sync_copy(acc_sh.at[pl.ds(sid * STRIPE, STRIPE)],
                    out_hbm.at[cid, pl.ds(sid * STRIPE, STRIPE)])


def _dinv_body(deg_ref, o_ref):
    deg = deg_ref[0] + deg_ref[1] + 1.0
    o_ref[...] = lax.rsqrt(deg)


_dinv_call = pl.pallas_call(
    _dinv_body,
    out_shape=jax.ShapeDtypeStruct((NR, 128), jnp.float32),
)

_BLK = 1000
_GRID = N // _BLK


def _scale_mm_body(x_ref, w_ref, dinv_ref, g_ref):
    h = jnp.dot(x_ref[...], w_ref[...], preferred_element_type=jnp.float32)
    g_ref[...] = h * dinv_ref[...]


_scale_mm_call = pl.pallas_call(
    _scale_mm_body,
    grid=(_GRID,),
    in_specs=[
        pl.BlockSpec((_BLK, F), lambda i: (i, 0)),
        pl.BlockSpec((F, F), lambda i: (0, 0)),
        pl.BlockSpec((_BLK, 1), lambda i: (i, 0)),
    ],
    out_specs=pl.BlockSpec((_BLK, F), lambda i: (i, 0)),
    out_shape=jax.ShapeDtypeStruct((N, F), jnp.float32),
)


def _layer_body(p_ref, g1_ref, dinv_ref, b1_ref, w2_ref, x1_ref, g2_ref):
    t = (p_ref[0] + p_ref[1] + g1_ref[...]) * dinv_ref[...] + b1_ref[...]
    x1 = jnp.maximum(t, 0.0)
    x1_ref[...] = x1
    h2 = jnp.dot(x1, w2_ref[...], preferred_element_type=jnp.float32)
    g2_ref[...] = h2 * dinv_ref[...]


_layer_call = pl.pallas_call(
    _layer_body,
    grid=(_GRID,),
    in_specs=[
        pl.BlockSpec((2, _BLK, F), lambda i: (0, i, 0)),
        pl.BlockSpec((_BLK, F), lambda i: (i, 0)),
        pl.BlockSpec((_BLK, 1), lambda i: (i, 0)),
        pl.BlockSpec((1, F), lambda i: (0, 0)),
        pl.BlockSpec((F, F), lambda i: (0, 0)),
    ],
    out_specs=[
        pl.BlockSpec((_BLK, F), lambda i: (i, 0)),
        pl.BlockSpec((_BLK, F), lambda i: (i, 0)),
    ],
    out_shape=[
        jax.ShapeDtypeStruct((N, F), jnp.float32),
        jax.ShapeDtypeStruct((N, F), jnp.float32),
    ],
)


def _final_body(q_ref, g2_ref, dinv_ref, b2_ref, x1_ref, wt_ref, wb_ref,
                bl_ref, o_ref):
    t = (q_ref[0] + q_ref[1] + g2_ref[...]) * dinv_ref[...] + b2_ref[...]
    x2 = jnp.maximum(t, 0.0)
    o_ref[...] = (
        jnp.dot(x1_ref[...], wt_ref[...], preferred_element_type=jnp.float32)
        + jnp.dot(x2, wb_ref[...], preferred_element_type=jnp.float32)
        + bl_ref[...]
    )


_final_call = pl.pallas_call(
    _final_body,
    grid=(_GRID,),
    in_specs=[
        pl.BlockSpec((2, _BLK, F), lambda i: (0, i, 0)),
        pl.BlockSpec((_BLK, F), lambda i: (i, 0)),
        pl.BlockSpec((_BLK, 1), lambda i: (i, 0)),
        pl.BlockSpec((1, F), lambda i: (0, 0)),
        pl.BlockSpec((_BLK, F), lambda i: (i, 0)),
        pl.BlockSpec((F, F), lambda i: (0, 0)),
        pl.BlockSpec((F, F), lambda i: (0, 0)),
        pl.BlockSpec((1, F), lambda i: (0, 0)),
    ],
    out_specs=pl.BlockSpec((_BLK, F), lambda i: (i, 0)),
    out_shape=jax.ShapeDtypeStruct((N, F), jnp.float32),
)


def kernel(x, edge_index, W1, b1, W2, b2, W_lin, b_lin):
    src = edge_index[0].astype(jnp.int32)
    dst = edge_index[1].astype(jnp.int32)
    ppw = (EP - E) // NW
    rpw = E // NW
    src_pad = jnp.zeros((NW, ppw), jnp.int32)
    dst_pad = jnp.broadcast_to(N + jnp.arange(ppw, dtype=jnp.int32), (NW, ppw))
    src3 = jnp.concatenate([src.reshape(NW, rpw), src_pad], axis=1)
    dst3 = jnp.concatenate([dst.reshape(NW, rpw), dst_pad], axis=1)
    src3 = src3.reshape(NW, GW, GCH, 128)
    dst3 = dst3.reshape(NW, GW, GCH, 128)
    zeros2d = jnp.zeros((AR, F), jnp.float32)
    zeros_deg = jnp.zeros((AR,), jnp.float32)

    deg_parts = _deg_kernel(dst3, zeros_deg)
    dinv2d = _dinv_call(deg_parts.reshape(2, NR, 128))
    dinv_col = dinv2d.reshape(-1)[:N, None]

    g1 = _scale_mm_call(x, W1, dinv_col)
    p = _agg_kernel(g1, src3, dst3, zeros2d)
    x1, g2 = _layer_call(p, g1, dinv_col, b1.reshape(1, F), W2)
    q = _agg_kernel(g2, src3, dst3, zeros2d)
    out = _final_call(q, g2, dinv_col, b2.reshape(1, F), x1,
                      W_lin[:F], W_lin[F:], b_lin.reshape(1, F))
    return out
